# baseline (device time: 185851 ns/iter reference)
import jax
import jax.numpy as jnp
from jax import lax
from jax.experimental import pallas as pl
from jax.experimental.pallas import tpu as pltpu

N_DEV = 16
S = 4

RING = [0, 4, 8, 12, 13, 9, 5, 1, 2, 6, 10, 14, 15, 11, 7, 3]
POS = [0] * N_DEV
for _p, _d in enumerate(RING):
    POS[_d] = _p
NEXT = [RING[(POS[d] + 1) % N_DEV] for d in range(N_DEV)]
PREV = [RING[(POS[d] - 1) % N_DEV] for d in range(N_DEV)]


def kernel(x, w_mat):
    m_per, k = x.shape
    _, n_per = w_mat.shape
    m_tot = m_per * N_DEV
    kq = k // 4

    def body(x_ref, w_ref, o_ref, xbf, bufs, ssems, rsems,
             asend, arecv, a_ssem, a_rsem, credits):
        my = lax.axis_index("i")

        def lut(table, idx):
            out = jnp.int32(table[0])
            for j in range(1, N_DEV):
                out = jnp.where(idx == j, jnp.int32(table[j]), out)
            return out

        pos = lut(POS, my)
        right = lut(NEXT, my)
        left = lut(PREV, my)

        bar = pltpu.get_barrier_semaphore()
        for nbr in (left, right):
            pl.semaphore_signal(bar, inc=1, device_id=(nbr,),
                                device_id_type=pl.DeviceIdType.MESH)
        for q in range(4):
            xbf[q, :, :] = x_ref[:, q * kq:(q + 1) * kq].astype(jnp.bfloat16)
        pl.semaphore_wait(bar, 2)

        def origin_cw(h):
            return lut(RING, lax.rem(pos + (N_DEV - h), N_DEV))

        def origin_ccw(h):
            return lut(RING, lax.rem(pos + h, N_DEV))

        HOPS = [8, 8, 7, 7, 7, 7, 8, 8]
        DST = [right] * 4 + [left] * 4
        UP = [left] * 4 + [right] * 4

        def desc(p, h):
            q = p % 4
            src = xbf.at[q] if h == 1 else bufs.at[p, (h - 2) % S]
            return pltpu.make_async_remote_copy(
                src_ref=src, dst_ref=bufs.at[p, (h - 1) % S],
                send_sem=ssems.at[p, (h - 1) % S],
                recv_sem=rsems.at[p, (h - 1) % S],
                device_id=(DST[p],), device_id_type=pl.DeviceIdType.MESH)

        def step(p, h):
            desc(p, h).wait_recv()
            if h < HOPS[p]:
                if h + 1 > S:
                    pl.semaphore_wait(credits.at[p], 1)
                desc(p, h + 1).start()
            desc(p, h).wait_send()
            if 2 <= h <= HOPS[p] - S + 1:
                pl.semaphore_signal(credits.at[p], inc=1,
                                    device_id=(UP[p],),
                                    device_id_type=pl.DeviceIdType.MESH)

        ORDER = [0, 6, 1, 7, 2, 4, 3, 5]
        for p in ORDER:
            desc(p, 1).start()

        arecv[...] = jnp.zeros(arecv.shape, arecv.dtype)
        wbf = w_ref[...].astype(jnp.bfloat16)
        wq = [wbf[q * kq:(q + 1) * kq, :] for q in range(4)]

        def qdot(buf_p, s, q):
            return jnp.dot(bufs[buf_p, s], wq[q],
                           preferred_element_type=jnp.float32)

        blk = sum(jnp.dot(xbf[q], wq[q], preferred_element_type=jnp.float32)
                  for q in range(4))
        o_ref[pl.ds(my * m_per, m_per), :] = blk
        amax = jnp.max(jnp.abs(blk))

        for h in range(1, 9):
            s = (h - 1) % S
            if h <= 7:
                step(0, h)
                acc_l = qdot(0, s, 0)
                step(6, h)
                acc_r = qdot(6, s, 2)
                step(1, h)
                acc_l = acc_l + qdot(1, s, 1)
                step(7, h)
                acc_r = acc_r + qdot(7, s, 3)
                step(2, h)
                acc_l = acc_l + qdot(2, s, 2)
                step(4, h)
                acc_r = acc_r + qdot(4, s, 0)
                step(3, h)
                acc_l = acc_l + qdot(3, s, 3)
                o_ref[pl.ds(origin_cw(h) * m_per, m_per), :] = acc_l
                amax = jnp.maximum(amax, jnp.max(jnp.abs(acc_l)))
                step(5, h)
                acc_r = acc_r + qdot(5, s, 1)
                o_ref[pl.ds(origin_ccw(h) * m_per, m_per), :] = acc_r
                amax = jnp.maximum(amax, jnp.max(jnp.abs(acc_r)))
            else:
                step(0, h)
                acc = qdot(0, s, 0)
                step(6, h)
                acc = acc + qdot(6, s, 2)
                step(1, h)
                acc = acc + qdot(1, s, 1)
                step(7, h)
                acc = acc + qdot(7, s, 3)
                o_ref[pl.ds(origin_cw(8) * m_per, m_per), :] = acc
                amax = jnp.maximum(amax, jnp.max(jnp.abs(acc)))

        asend[...] = amax * jnp.ones(asend.shape, jnp.float32)
        for o in range(1, N_DEV):
            d = lax.rem(my + o, N_DEV)
            snd = pltpu.make_async_remote_copy(
                src_ref=asend, dst_ref=arecv.at[my],
                send_sem=a_ssem.at[o - 1], recv_sem=a_rsem.at[my],
                device_id=(d,), device_id_type=pl.DeviceIdType.MESH)
            snd.start()
        for o in range(1, N_DEV):
            sl = lax.rem(my + o, N_DEV)
            rcv = pltpu.make_async_remote_copy(
                src_ref=asend, dst_ref=arecv.at[sl],
                send_sem=a_ssem.at[o - 1], recv_sem=a_rsem.at[sl],
                device_id=(my,), device_id_type=pl.DeviceIdType.MESH)
            rcv.wait_recv()
        for o in range(1, N_DEV):
            snd_w = pltpu.make_async_remote_copy(
                src_ref=asend, dst_ref=arecv.at[my],
                send_sem=a_ssem.at[o - 1], recv_sem=a_rsem.at[my],
                device_id=(my,), device_id_type=pl.DeviceIdType.MESH)
            snd_w.wait_send()
        g = jnp.maximum(jnp.max(arecv[...]), amax)

        inv = 448.0 / g
        scale = g / 448.0
        y = o_ref[...]
        q8 = jnp.clip(y * inv, -448.0, 448.0).astype(jnp.float8_e4m3fn)
        o_ref[...] = q8.astype(jnp.float32) * scale

    return pl.pallas_call(
        body,
        out_shape=jax.ShapeDtypeStruct((m_tot, n_per), jnp.float32),
        in_specs=[pl.BlockSpec(memory_space=pltpu.VMEM),
                  pl.BlockSpec(memory_space=pltpu.VMEM)],
        out_specs=pl.BlockSpec(memory_space=pltpu.VMEM),
        scratch_shapes=[
            pltpu.VMEM((4, m_per, kq), jnp.bfloat16),
            pltpu.VMEM((8, S, m_per, kq), jnp.bfloat16),
            pltpu.SemaphoreType.DMA((8, S)),
            pltpu.SemaphoreType.DMA((8, S)),
            pltpu.VMEM((8, 128), jnp.float32),
            pltpu.VMEM((N_DEV, 8, 128), jnp.float32),
            pltpu.SemaphoreType.DMA((N_DEV - 1,)),
            pltpu.SemaphoreType.DMA((N_DEV,)),
            pltpu.SemaphoreType.REGULAR((8,)),
        ],
        compiler_params=pltpu.CompilerParams(collective_id=0),
    )(x, w_mat)


# device time: 175639 ns/iter; 1.0581x vs baseline; 1.0581x over previous
import jax
import jax.numpy as jnp
from jax import lax
from jax.experimental import pallas as pl
from jax.experimental.pallas import tpu as pltpu

N_DEV = 16
S = 3

RING = [0, 4, 8, 12, 13, 9, 5, 1, 2, 6, 10, 14, 15, 11, 7, 3]
POS = [0] * N_DEV
for _p, _d in enumerate(RING):
    POS[_d] = _p
NEXT = [RING[(POS[d] + 1) % N_DEV] for d in range(N_DEV)]
PREV = [RING[(POS[d] - 1) % N_DEV] for d in range(N_DEV)]

PARTNER = [1, 0, 3, 2, 5, 4, 7, 6, 9, 8, 11, 10, 15, 14, 13, 12]
RELAY_DIR = [0] * N_DEV
RELAY_HOP = [0] * N_DEV
for _me in range(N_DEV):
    _c = PARTNER[_me]
    _a = RING[(POS[_c] + 8) % N_DEV]
    _dcw = (POS[_me] - POS[_a]) % N_DEV
    if 1 <= _dcw <= 7:
        RELAY_DIR[_me], RELAY_HOP[_me] = 0, _dcw
    else:
        RELAY_DIR[_me], RELAY_HOP[_me] = 1, N_DEV - _dcw
assert all(1 <= h <= 7 for h in RELAY_HOP)


def kernel(x, w_mat):
    m_per, k = x.shape
    _, n_per = w_mat.shape
    m_tot = m_per * N_DEV
    kq = k // 4

    def body(x_ref, w_ref, o_ref, xbf, bufs, ssems, rsems,
             rstage, rbuf, r_ssem, r_rsem,
             asend, arecv, a_ssem, a_rsem, credits):
        my = lax.axis_index("i")

        def lut(table, idx):
            out = jnp.int32(table[0])
            for j in range(1, N_DEV):
                out = jnp.where(idx == j, jnp.int32(table[j]), out)
            return out

        pos = lut(POS, my)
        right = lut(NEXT, my)
        left = lut(PREV, my)
        partner = lut(PARTNER, my)
        relay_dir = lut(RELAY_DIR, my)
        relay_hop = lut(RELAY_HOP, my)

        bar = pltpu.get_barrier_semaphore()
        for nbr in (left, right, partner):
            pl.semaphore_signal(bar, inc=1, device_id=(nbr,),
                                device_id_type=pl.DeviceIdType.MESH)
        for q in range(4):
            xbf[q, :, :] = x_ref[:, q * kq:(q + 1) * kq].astype(jnp.bfloat16)
        pl.semaphore_wait(bar, 3)

        def origin_cw(h):
            return lut(RING, lax.rem(pos + (N_DEV - h), N_DEV))

        def origin_ccw(h):
            return lut(RING, lax.rem(pos + h, N_DEV))

        DST = [right] * 4 + [left] * 4
        UP = [left] * 4 + [right] * 4

        def desc(p, h):
            q = p % 4
            src = xbf.at[q] if h == 1 else bufs.at[p, (h - 2) % S]
            return pltpu.make_async_remote_copy(
                src_ref=src, dst_ref=bufs.at[p, (h - 1) % S],
                send_sem=ssems.at[p, (h - 1) % S],
                recv_sem=rsems.at[p, (h - 1) % S],
                device_id=(DST[p],), device_id_type=pl.DeviceIdType.MESH)

        def rdesc():
            return pltpu.make_async_remote_copy(
                src_ref=rstage, dst_ref=rbuf,
                send_sem=r_ssem.at[0], recv_sem=r_rsem.at[0],
                device_id=(partner,), device_id_type=pl.DeviceIdType.MESH)

        def step(p, h):
            desc(p, h).wait_recv()
            if h < 7:
                if h + 1 > S:
                    pl.semaphore_wait(credits.at[p], 1)
                desc(p, h + 1).start()
            desc(p, h).wait_send()
            if 2 <= h <= 7 - S + 1:
                pl.semaphore_signal(credits.at[p], inc=1,
                                    device_id=(UP[p],),
                                    device_id_type=pl.DeviceIdType.MESH)

        ORDER = [0, 6, 1, 7, 2, 4, 3, 5]
        for p in ORDER:
            desc(p, 1).start()

        arecv[...] = jnp.zeros(arecv.shape, arecv.dtype)
        wbf = w_ref[...].astype(jnp.bfloat16)
        wq = [wbf[q * kq:(q + 1) * kq, :] for q in range(4)]

        def qdot(buf_p, s, q):
            return jnp.dot(bufs[buf_p, s], wq[q],
                           preferred_element_type=jnp.float32)

        blk = sum(jnp.dot(xbf[q], wq[q], preferred_element_type=jnp.float32)
                  for q in range(4))
        o_ref[pl.ds(my * m_per, m_per), :] = blk
        amax = jnp.max(jnp.abs(blk))

        for h in range(1, 8):
            s = (h - 1) % S
            step(0, h)
            acc_l = qdot(0, s, 0)
            step(6, h)
            acc_r = qdot(6, s, 2)
            step(1, h)
            acc_l = acc_l + qdot(1, s, 1)
            step(7, h)
            acc_r = acc_r + qdot(7, s, 3)
            step(2, h)
            acc_l = acc_l + qdot(2, s, 2)
            step(4, h)
            acc_r = acc_r + qdot(4, s, 0)
            step(3, h)
            acc_l = acc_l + qdot(3, s, 3)
            o_ref[pl.ds(origin_cw(h) * m_per, m_per), :] = acc_l
            amax = jnp.maximum(amax, jnp.max(jnp.abs(acc_l)))
            step(5, h)
            acc_r = acc_r + qdot(5, s, 1)
            o_ref[pl.ds(origin_ccw(h) * m_per, m_per), :] = acc_r
            amax = jnp.maximum(amax, jnp.max(jnp.abs(acc_r)))

            @pl.when(h == relay_hop)
            def _():
                for q in range(4):
                    rstage[q, :, :] = jnp.where(relay_dir == 0,
                                                bufs[q, s],
                                                bufs[4 + q, s])
                rdesc().start()

        rdesc().wait_send()
        rdesc().wait_recv()
        blk = sum(jnp.dot(rbuf[q], wq[q], preferred_element_type=jnp.float32)
                  for q in range(4))
        o_ref[pl.ds(origin_cw(8) * m_per, m_per), :] = blk
        amax = jnp.maximum(amax, jnp.max(jnp.abs(blk)))

        asend[...] = amax * jnp.ones(asend.shape, jnp.float32)
        for o in range(1, N_DEV):
            d = lax.rem(my + o, N_DEV)
            snd = pltpu.make_async_remote_copy(
                src_ref=asend, dst_ref=arecv.at[my],
                send_sem=a_ssem.at[o - 1], recv_sem=a_rsem.at[my],
                device_id=(d,), device_id_type=pl.DeviceIdType.MESH)
            snd.start()
        for o in range(1, N_DEV):
            sl = lax.rem(my + o, N_DEV)
            rcv = pltpu.make_async_remote_copy(
                src_ref=asend, dst_ref=arecv.at[sl],
                send_sem=a_ssem.at[o - 1], recv_sem=a_rsem.at[sl],
                device_id=(my,), device_id_type=pl.DeviceIdType.MESH)
            rcv.wait_recv()
        for o in range(1, N_DEV):
            snd_w = pltpu.make_async_remote_copy(
                src_ref=asend, dst_ref=arecv.at[my],
                send_sem=a_ssem.at[o - 1], recv_sem=a_rsem.at[my],
                device_id=(my,), device_id_type=pl.DeviceIdType.MESH)
            snd_w.wait_send()
        g = jnp.maximum(jnp.max(arecv[...]), amax)

        inv = 448.0 / g
        scale = g / 448.0
        y = o_ref[...]
        q8 = jnp.clip(y * inv, -448.0, 448.0).astype(jnp.float8_e4m3fn)
        o_ref[...] = q8.astype(jnp.float32) * scale

    return pl.pallas_call(
        body,
        out_shape=jax.ShapeDtypeStruct((m_tot, n_per), jnp.float32),
        in_specs=[pl.BlockSpec(memory_space=pltpu.VMEM),
                  pl.BlockSpec(memory_space=pltpu.VMEM)],
        out_specs=pl.BlockSpec(memory_space=pltpu.VMEM),
        scratch_shapes=[
            pltpu.VMEM((4, m_per, kq), jnp.bfloat16),
            pltpu.VMEM((8, S, m_per, kq), jnp.bfloat16),
            pltpu.SemaphoreType.DMA((8, S)),
            pltpu.SemaphoreType.DMA((8, S)),
            pltpu.VMEM((4, m_per, kq), jnp.bfloat16),
            pltpu.VMEM((4, m_per, kq), jnp.bfloat16),
            pltpu.SemaphoreType.DMA((1,)),
            pltpu.SemaphoreType.DMA((1,)),
            pltpu.VMEM((8, 128), jnp.float32),
            pltpu.VMEM((N_DEV, 8, 128), jnp.float32),
            pltpu.SemaphoreType.DMA((N_DEV - 1,)),
            pltpu.SemaphoreType.DMA((N_DEV,)),
            pltpu.SemaphoreType.REGULAR((8,)),
        ],
        compiler_params=pltpu.CompilerParams(collective_id=0),
    )(x, w_mat)


# device time: 167082 ns/iter; 1.1123x vs baseline; 1.0512x over previous
import jax
import jax.numpy as jnp
from jax import lax
from jax.experimental import pallas as pl
from jax.experimental.pallas import tpu as pltpu

N_DEV = 16
S = 3

RING = [0, 4, 8, 12, 13, 9, 5, 1, 2, 6, 10, 14, 15, 11, 7, 3]
POS = [0] * N_DEV
for _p, _d in enumerate(RING):
    POS[_d] = _p
NEXT = [RING[(POS[d] + 1) % N_DEV] for d in range(N_DEV)]
PREV = [RING[(POS[d] - 1) % N_DEV] for d in range(N_DEV)]

PARTNER = [1, 0, 3, 2, 5, 4, 7, 6, 9, 8, 11, 10, 15, 14, 13, 12]
RD = [0] * N_DEV
AH = [0] * N_DEV
BH = [0] * N_DEV
CH = [0] * N_DEV
for _me in range(N_DEV):
    _delta = (POS[PARTNER[_me]] - POS[_me]) % N_DEV
    if _delta < 8:
        RD[_me] = 0
        AH[_me], BH[_me], CH[_me] = 8 - _delta, 7 - _delta, 9 - _delta
    else:
        RD[_me] = 1
        AH[_me], BH[_me], CH[_me] = _delta - 8, _delta - 7, (_delta + 7) % N_DEV


def kernel(x, w_mat):
    m_per, k = x.shape
    _, n_per = w_mat.shape
    m_tot = m_per * N_DEV
    kq = k // 4

    def body(x_ref, w_ref, o_ref, xbf, bufs, ssems, rsems,
             rstage, rbuf, r_ssem, r_rsem,
             asend, arecv, a_ssem, a_rsem, credits):
        my = lax.axis_index("i")

        def lut(table, idx):
            out = jnp.int32(table[0])
            for j in range(1, N_DEV):
                out = jnp.where(idx == j, jnp.int32(table[j]), out)
            return out

        pos = lut(POS, my)
        right = lut(NEXT, my)
        left = lut(PREV, my)
        partner = lut(PARTNER, my)
        rd = lut(RD, my)
        ah = lut(AH, my)
        bh = lut(BH, my)
        ch = lut(CH, my)

        bar = pltpu.get_barrier_semaphore()
        for nbr in (left, right, partner):
            pl.semaphore_signal(bar, inc=1, device_id=(nbr,),
                                device_id_type=pl.DeviceIdType.MESH)
        for q in range(4):
            xbf[q, :, :] = x_ref[:, q * kq:(q + 1) * kq].astype(jnp.bfloat16)
        pl.semaphore_wait(bar, 3)

        def origin_cw(h):
            return lut(RING, lax.rem(pos + (N_DEV - h), N_DEV))

        def origin_ccw(h):
            return lut(RING, lax.rem(pos + h, N_DEV))

        HOPS = [7, 7, 6, 6, 6, 6, 7, 7]
        DST = [right] * 4 + [left] * 4
        UP = [left] * 4 + [right] * 4

        def desc(p, h):
            q = p % 4
            src = xbf.at[q] if h == 1 else bufs.at[p, (h - 2) % S]
            return pltpu.make_async_remote_copy(
                src_ref=src, dst_ref=bufs.at[p, (h - 1) % S],
                send_sem=ssems.at[p, (h - 1) % S],
                recv_sem=rsems.at[p, (h - 1) % S],
                device_id=(DST[p],), device_id_type=pl.DeviceIdType.MESH)

        RSEC = [(0, 4), (4, 2), (6, 2)]

        def rdesc(i):
            o, n = RSEC[i]
            return pltpu.make_async_remote_copy(
                src_ref=rstage.at[pl.ds(o, n)], dst_ref=rbuf.at[pl.ds(o, n)],
                send_sem=r_ssem.at[i], recv_sem=r_rsem.at[i],
                device_id=(partner,), device_id_type=pl.DeviceIdType.MESH)

        def step(p, h):
            desc(p, h).wait_recv()
            if h < HOPS[p]:
                if h + 1 > S:
                    pl.semaphore_wait(credits.at[p], 1)
                desc(p, h + 1).start()
            desc(p, h).wait_send()
            if 2 <= h <= HOPS[p] - S + 1:
                pl.semaphore_signal(credits.at[p], inc=1,
                                    device_id=(UP[p],),
                                    device_id_type=pl.DeviceIdType.MESH)

        ORDER = [0, 6, 1, 7, 2, 4, 3, 5]
        for p in ORDER:
            desc(p, 1).start()

        @pl.when(bh == 0)
        def _():
            rstage[4, :, :] = xbf[2]
            rstage[5, :, :] = xbf[3]
            rdesc(1).start()

        @pl.when(ch == 0)
        def _():
            rstage[6, :, :] = xbf[0]
            rstage[7, :, :] = xbf[1]
            rdesc(2).start()

        arecv[...] = jnp.zeros(arecv.shape, arecv.dtype)
        wbf = w_ref[...].astype(jnp.bfloat16)
        wq = [wbf[q * kq:(q + 1) * kq, :] for q in range(4)]

        def qdot(buf_p, s, q):
            return jnp.dot(bufs[buf_p, s], wq[q],
                           preferred_element_type=jnp.float32)

        blk = sum(jnp.dot(xbf[q], wq[q], preferred_element_type=jnp.float32)
                  for q in range(4))
        o_ref[pl.ds(my * m_per, m_per), :] = blk
        amax = jnp.max(jnp.abs(blk))

        for h in range(1, 8):
            s = (h - 1) % S
            step(0, h)
            acc_l = qdot(0, s, 0)
            step(6, h)
            acc_r = qdot(6, s, 2)
            step(1, h)
            acc_l = acc_l + qdot(1, s, 1)
            step(7, h)
            acc_r = acc_r + qdot(7, s, 3)
            if h <= 6:
                step(2, h)
                acc_l = acc_l + qdot(2, s, 2)
                step(4, h)
                acc_r = acc_r + qdot(4, s, 0)
                step(3, h)
                acc_l = acc_l + qdot(3, s, 3)
                o_ref[pl.ds(origin_cw(h) * m_per, m_per), :] = acc_l
                amax = jnp.maximum(amax, jnp.max(jnp.abs(acc_l)))
                step(5, h)
                acc_r = acc_r + qdot(5, s, 1)
                o_ref[pl.ds(origin_ccw(h) * m_per, m_per), :] = acc_r
                amax = jnp.maximum(amax, jnp.max(jnp.abs(acc_r)))
            else:
                rdesc(1).wait_recv()
                acc_l = (acc_l + jnp.dot(rbuf[4], wq[2],
                                         preferred_element_type=jnp.float32)
                         + jnp.dot(rbuf[5], wq[3],
                                   preferred_element_type=jnp.float32))
                o_ref[pl.ds(origin_cw(h) * m_per, m_per), :] = acc_l
                amax = jnp.maximum(amax, jnp.max(jnp.abs(acc_l)))
                rdesc(2).wait_recv()
                acc_r = (acc_r + jnp.dot(rbuf[6], wq[0],
                                         preferred_element_type=jnp.float32)
                         + jnp.dot(rbuf[7], wq[1],
                                   preferred_element_type=jnp.float32))
                o_ref[pl.ds(origin_ccw(h) * m_per, m_per), :] = acc_r
                amax = jnp.maximum(amax, jnp.max(jnp.abs(acc_r)))

            @pl.when(h == ah)
            def _():
                for q in range(4):
                    rstage[q, :, :] = jnp.where(rd == 0,
                                                bufs[q, s], bufs[4 + q, s])
                rdesc(0).start()

            if h <= 6:
                @pl.when(h == bh)
                def _():
                    rstage[4, :, :] = jnp.where(rd == 0,
                                                bufs[2, s], bufs[6, s])
                    rstage[5, :, :] = jnp.where(rd == 0,
                                                bufs[3, s], bufs[7, s])
                    rdesc(1).start()

                @pl.when(h == ch)
                def _():
                    rstage[6, :, :] = jnp.where(rd == 0,
                                                bufs[0, s], bufs[4, s])
                    rstage[7, :, :] = jnp.where(rd == 0,
                                                bufs[1, s], bufs[5, s])
                    rdesc(2).start()

        rdesc(0).wait_recv()
        blk = sum(jnp.dot(rbuf[q], wq[q], preferred_element_type=jnp.float32)
                  for q in range(4))
        o_ref[pl.ds(origin_cw(8) * m_per, m_per), :] = blk
        amax = jnp.maximum(amax, jnp.max(jnp.abs(blk)))
        for i in range(3):
            rdesc(i).wait_send()

        asend[...] = amax * jnp.ones(asend.shape, jnp.float32)
        for o in range(1, N_DEV):
            d = lax.rem(my + o, N_DEV)
            snd = pltpu.make_async_remote_copy(
                src_ref=asend, dst_ref=arecv.at[my],
                send_sem=a_ssem.at[o - 1], recv_sem=a_rsem.at[my],
                device_id=(d,), device_id_type=pl.DeviceIdType.MESH)
            snd.start()
        for o in range(1, N_DEV):
            sl = lax.rem(my + o, N_DEV)
            rcv = pltpu.make_async_remote_copy(
                src_ref=asend, dst_ref=arecv.at[sl],
                send_sem=a_ssem.at[o - 1], recv_sem=a_rsem.at[sl],
                device_id=(my,), device_id_type=pl.DeviceIdType.MESH)
            rcv.wait_recv()
        for o in range(1, N_DEV):
            snd_w = pltpu.make_async_remote_copy(
                src_ref=asend, dst_ref=arecv.at[my],
                send_sem=a_ssem.at[o - 1], recv_sem=a_rsem.at[my],
                device_id=(my,), device_id_type=pl.DeviceIdType.MESH)
            snd_w.wait_send()
        g = jnp.maximum(jnp.max(arecv[...]), amax)

        inv = 448.0 / g
        scale = g / 448.0
        y = o_ref[...]
        q8 = jnp.clip(y * inv, -448.0, 448.0).astype(jnp.float8_e4m3fn)
        o_ref[...] = q8.astype(jnp.float32) * scale

    return pl.pallas_call(
        body,
        out_shape=jax.ShapeDtypeStruct((m_tot, n_per), jnp.float32),
        in_specs=[pl.BlockSpec(memory_space=pltpu.VMEM),
                  pl.BlockSpec(memory_space=pltpu.VMEM)],
        out_specs=pl.BlockSpec(memory_space=pltpu.VMEM),
        scratch_shapes=[
            pltpu.VMEM((4, m_per, kq), jnp.bfloat16),
            pltpu.VMEM((8, S, m_per, kq), jnp.bfloat16),
            pltpu.SemaphoreType.DMA((8, S)),
            pltpu.SemaphoreType.DMA((8, S)),
            pltpu.VMEM((8, m_per, kq), jnp.bfloat16),
            pltpu.VMEM((8, m_per, kq), jnp.bfloat16),
            pltpu.SemaphoreType.DMA((3,)),
            pltpu.SemaphoreType.DMA((3,)),
            pltpu.VMEM((8, 128), jnp.float32),
            pltpu.VMEM((N_DEV, 8, 128), jnp.float32),
            pltpu.SemaphoreType.DMA((N_DEV - 1,)),
            pltpu.SemaphoreType.DMA((N_DEV,)),
            pltpu.SemaphoreType.REGULAR((8,)),
        ],
        compiler_params=pltpu.CompilerParams(collective_id=0),
    )(x, w_mat)
